# baseline (speedup 1.0000x reference)
from functools import partial

import numpy as np
import jax
import jax.numpy as jnp
from jax.experimental import pallas as pl
from jax.experimental.pallas import tpu as pltpu
from jax.experimental.shard_map import shard_map

_VMEM = 60 * 1024 * 1024
_DT = jnp.bfloat16


def _conv1_pool_kernel(x_ref, w_ref, b_ref, o_ref):
    bt = x_ref.shape[2]
    offs = [kh * 32 + kw for kh in range(3) for kw in range(3)]
    lo, hi = [], []
    for g in range(5):
        pieces = [x_ref[c, 480 * h + o + g * 96:480 * h + o + g * 96 + 96]
                  for h in range(2) for o in offs for c in range(3)]
        xg = jnp.transpose(jnp.stack(pieces, axis=1), (0, 2, 1))
        xs = xg.reshape(96 * bt, 54)
        y = jnp.dot(xs, w_ref[...], preferred_element_type=jnp.float32)
        y = jnp.maximum(y + b_ref[...], 0.0).reshape(3, 32, bt, 256)
        y = jnp.maximum(jnp.maximum(y[0], y[1]), y[2])
        y = y[:30].reshape(10, 3, bt, 256)
        y = jnp.maximum(jnp.maximum(y[:, 0], y[:, 1]), y[:, 2])
        lo.append(y[..., :128])
        hi.append(y[..., 128:])
    o_ref[...] = jnp.concatenate(lo + hi, axis=0).astype(o_ref.dtype)


def _stage2_kernel(x_ref, w2_ref, b2_ref, w3_ref, b3_ref, wh_ref, bh_ref,
                   o_ref):
    bt = x_ref.shape[1]
    offs = [(t // 3) * 10 + t % 3 for t in range(9)]
    xs2 = jnp.concatenate(
        [x_ref[o:o + 78].reshape(78 * bt, 128) for o in offs], axis=1)
    acc2 = jnp.dot(xs2, w2_ref[...], preferred_element_type=jnp.float32)
    y2 = jnp.maximum(acc2 + b2_ref[...], 0.0).astype(x_ref.dtype)
    y2 = y2.reshape(78, bt, 256)

    xs3 = jnp.concatenate(
        [y2[o:o + 56].reshape(56 * bt, 256) for o in offs], axis=1)
    acc3 = jnp.dot(xs3, w3_ref[...], preferred_element_type=jnp.float32)
    y3 = jnp.maximum(acc3 + b3_ref[...], 0.0).astype(x_ref.dtype)
    y3 = y3.reshape(56, bt, 384)

    feats = jnp.concatenate(
        [y3[oh * 10 + ow] for oh in range(6) for ow in range(6)], axis=1)
    out = jnp.dot(feats, wh_ref[...], preferred_element_type=jnp.float32)
    o_ref[...] = out + bh_ref[...]


def _forward(x_nchw, conv1_w, conv1_b, conv2_w, conv2_b, conv3_w, conv3_b,
             head_w, head_b):
    B = x_nchw.shape[0]
    bt1 = 128 if B % 128 == 0 else B
    bt2 = 64 if B % 64 == 0 else B

    xt = jnp.transpose(x_nchw.reshape(B, 3 * 1024).astype(_DT))
    x1 = jnp.pad(xt.reshape(3, 1024, B), ((0, 0), (0, 2), (0, 0)))

    w1 = jnp.zeros((54, 256), jnp.float32)
    w1 = w1.at[:27, :128].set(conv1_w).at[27:, 128:].set(conv1_w).astype(_DT)
    b1 = jnp.concatenate([conv1_b, conv1_b], axis=1)

    pooled = pl.pallas_call(
        _conv1_pool_kernel,
        out_shape=jax.ShapeDtypeStruct((100, B, 128), _DT),
        grid=(B // bt1,),
        in_specs=[
            pl.BlockSpec((3, 1026, bt1), lambda i: (0, 0, i)),
            pl.BlockSpec((54, 256), lambda i: (0, 0)),
            pl.BlockSpec((1, 256), lambda i: (0, 0)),
        ],
        out_specs=pl.BlockSpec((100, bt1, 128), lambda i: (0, i, 0)),
        compiler_params=pltpu.CompilerParams(
            dimension_semantics=("parallel",), vmem_limit_bytes=_VMEM),
    )(x1, w1, b1)

    w2r = conv2_w.astype(_DT)
    w3r = jnp.pad(conv3_w, ((0, 0), (0, 84))).astype(_DT)
    b3p = jnp.pad(conv3_b, ((0, 0), (0, 84)))
    whr = jnp.pad(head_w.reshape(36, 300, 10),
                  ((0, 0), (0, 84), (0, 0))).reshape(36 * 384, 10).astype(_DT)

    out = pl.pallas_call(
        _stage2_kernel,
        out_shape=jax.ShapeDtypeStruct((B, 10), jnp.float32),
        grid=(B // bt2,),
        in_specs=[
            pl.BlockSpec((100, bt2, 128), lambda i: (0, i, 0)),
            pl.BlockSpec((1152, 256), lambda i: (0, 0)),
            pl.BlockSpec((1, 256), lambda i: (0, 0)),
            pl.BlockSpec((2304, 384), lambda i: (0, 0)),
            pl.BlockSpec((1, 384), lambda i: (0, 0)),
            pl.BlockSpec((36 * 384, 10), lambda i: (0, 0)),
            pl.BlockSpec((1, 10), lambda i: (0, 0)),
        ],
        out_specs=pl.BlockSpec((bt2, 10), lambda i: (i, 0)),
        compiler_params=pltpu.CompilerParams(
            dimension_semantics=("parallel",), vmem_limit_bytes=_VMEM),
    )(pooled, w2r, conv2_b, w3r, b3p, whr, head_b)
    return out


def kernel(x_nchw, conv1_w, conv1_b, conv2_w, conv2_b, conv3_w, conv3_b,
           head_w, head_b, sel1, sel2, sel3):
    B = x_nchw.shape[0]
    devs = jax.devices()
    args = (x_nchw, conv1_w, conv1_b, conv2_w, conv2_b, conv3_w, conv3_b,
            head_w, head_b)
    if len(devs) >= 2 and B % 256 == 0:
        P = jax.sharding.PartitionSpec
        mesh = jax.sharding.Mesh(np.asarray(devs[:2]), ("b",))
        f = shard_map(_forward, mesh=mesh,
                      in_specs=(P("b"),) + (P(),) * 8,
                      out_specs=P("b"), check_rep=False)
        return f(*args)
    return _forward(*args)


# speedup vs baseline: 1.0643x; 1.0643x over previous
from functools import partial

import numpy as np
import jax
import jax.numpy as jnp
from jax.experimental import pallas as pl
from jax.experimental.pallas import tpu as pltpu
from jax.experimental.shard_map import shard_map

_VMEM = 60 * 1024 * 1024
_DT = jnp.bfloat16


def _conv1_pool_kernel(x_ref, w_ref, b_ref, o_ref):
    bt = x_ref.shape[2]
    offs = [kh * 32 + kw for kh in range(3) for kw in range(3)]
    lo, hi = [], []
    for g in range(5):
        pieces = [x_ref[c, 480 * h + o + g * 96:480 * h + o + g * 96 + 96]
                  for h in range(2) for o in offs for c in range(3)]
        xg = jnp.transpose(jnp.stack(pieces, axis=1), (0, 2, 1))
        xs = xg.reshape(96 * bt, 54)
        y = jnp.dot(xs, w_ref[...], preferred_element_type=jnp.float32)
        y = jnp.maximum(y + b_ref[...], 0.0).reshape(3, 32, bt, 256)
        y = jnp.maximum(jnp.maximum(y[0], y[1]), y[2])
        y = y[:30].reshape(10, 3, bt, 256)
        y = jnp.maximum(jnp.maximum(y[:, 0], y[:, 1]), y[:, 2])
        lo.append(y[..., :128])
        hi.append(y[..., 128:])
    o_ref[...] = jnp.concatenate(lo + hi, axis=0).astype(o_ref.dtype)


def _stage2_kernel(x_ref, w2_ref, b2_ref, w3_ref, b3_ref, wh_ref, bh_ref,
                   o_ref):
    bt = x_ref.shape[1]
    offs = [(t // 3) * 10 + t % 3 for t in range(9)]
    xs2 = jnp.concatenate(
        [x_ref[o:o + 78].reshape(78 * bt, 128) for o in offs], axis=1)
    acc2 = jnp.dot(xs2, w2_ref[...], preferred_element_type=jnp.float32)
    y2 = jnp.maximum(acc2 + b2_ref[...], 0.0).astype(x_ref.dtype)
    y2 = y2.reshape(78, bt, 256)

    xs3 = jnp.concatenate(
        [y2[o:o + 56].reshape(56 * bt, 256) for o in offs], axis=1)
    acc3 = jnp.dot(xs3, w3_ref[...], preferred_element_type=jnp.float32)
    y3 = jnp.maximum(acc3 + b3_ref[...], 0.0).astype(x_ref.dtype)
    y3 = y3.reshape(56, bt, 384)

    feats = jnp.concatenate(
        [y3[oh * 10 + ow] for oh in range(6) for ow in range(6)], axis=1)
    out = jnp.dot(feats, wh_ref[...], preferred_element_type=jnp.float32)
    o_ref[...] = out + bh_ref[...]


def _forward(xt, conv1_w, conv1_b, conv2_w, conv2_b, conv3_w, conv3_b,
             head_w, head_b):
    B = xt.shape[2]
    bt1 = 128 if B % 128 == 0 else B
    bt2 = 64 if B % 64 == 0 else B

    x1 = jnp.pad(xt, ((0, 0), (0, 2), (0, 0)))

    w1 = jnp.zeros((54, 256), jnp.float32)
    w1 = w1.at[:27, :128].set(conv1_w).at[27:, 128:].set(conv1_w).astype(_DT)
    b1 = jnp.concatenate([conv1_b, conv1_b], axis=1)

    pooled = pl.pallas_call(
        _conv1_pool_kernel,
        out_shape=jax.ShapeDtypeStruct((100, B, 128), _DT),
        grid=(B // bt1,),
        in_specs=[
            pl.BlockSpec((3, 1026, bt1), lambda i: (0, 0, i)),
            pl.BlockSpec((54, 256), lambda i: (0, 0)),
            pl.BlockSpec((1, 256), lambda i: (0, 0)),
        ],
        out_specs=pl.BlockSpec((100, bt1, 128), lambda i: (0, i, 0)),
        compiler_params=pltpu.CompilerParams(
            dimension_semantics=("parallel",), vmem_limit_bytes=_VMEM),
    )(x1, w1, b1)

    w2r = conv2_w.astype(_DT)
    w3r = jnp.pad(conv3_w, ((0, 0), (0, 84))).astype(_DT)
    b3p = jnp.pad(conv3_b, ((0, 0), (0, 84)))
    whr = jnp.pad(head_w.reshape(36, 300, 10),
                  ((0, 0), (0, 84), (0, 0))).reshape(36 * 384, 10).astype(_DT)

    out = pl.pallas_call(
        _stage2_kernel,
        out_shape=jax.ShapeDtypeStruct((B, 10), jnp.float32),
        grid=(B // bt2,),
        in_specs=[
            pl.BlockSpec((100, bt2, 128), lambda i: (0, i, 0)),
            pl.BlockSpec((1152, 256), lambda i: (0, 0)),
            pl.BlockSpec((1, 256), lambda i: (0, 0)),
            pl.BlockSpec((2304, 384), lambda i: (0, 0)),
            pl.BlockSpec((1, 384), lambda i: (0, 0)),
            pl.BlockSpec((36 * 384, 10), lambda i: (0, 0)),
            pl.BlockSpec((1, 10), lambda i: (0, 0)),
        ],
        out_specs=pl.BlockSpec((bt2, 10), lambda i: (i, 0)),
        compiler_params=pltpu.CompilerParams(
            dimension_semantics=("parallel",), vmem_limit_bytes=_VMEM),
    )(pooled, w2r, conv2_b, w3r, b3p, whr, head_b)
    return out


def kernel(x_nchw, conv1_w, conv1_b, conv2_w, conv2_b, conv3_w, conv3_b,
           head_w, head_b, sel1, sel2, sel3):
    B = x_nchw.shape[0]
    devs = jax.devices()
    xt = jnp.transpose(x_nchw.reshape(B, 3 * 1024).astype(_DT))
    xt = xt.reshape(3, 1024, B)
    args = (xt, conv1_w, conv1_b, conv2_w, conv2_b, conv3_w, conv3_b,
            head_w, head_b)
    if len(devs) >= 2 and B % 256 == 0:
        P = jax.sharding.PartitionSpec
        mesh = jax.sharding.Mesh(np.asarray(devs[:2]), ("b",))
        f = shard_map(_forward, mesh=mesh,
                      in_specs=(P(None, None, "b"),) + (P(),) * 8,
                      out_specs=P("b"), check_rep=False)
        return f(*args)
    return _forward(*args)


# speedup vs baseline: 1.2426x; 1.1675x over previous
from functools import partial

import jax
import jax.numpy as jnp
from jax.experimental import pallas as pl
from jax.experimental.pallas import tpu as pltpu

_VMEM = 60 * 1024 * 1024
_DT = jnp.bfloat16


def _conv1_pool_kernel(x_ref, w_ref, b_ref, o_ref):
    bt = x_ref.shape[2]
    offs = [kh * 32 + kw for kh in range(3) for kw in range(3)]
    lo, hi = [], []
    for g in range(5):
        pieces = [x_ref[c, 480 * h + o + g * 96:480 * h + o + g * 96 + 96]
                  for h in range(2) for o in offs for c in range(3)]
        xg = jnp.transpose(jnp.stack(pieces, axis=1), (0, 2, 1))
        xs = xg.reshape(96 * bt, 54)
        y = jnp.dot(xs, w_ref[...], preferred_element_type=jnp.float32)
        y = jnp.maximum(y + b_ref[...], 0.0).reshape(3, 32, bt, 256)
        y = jnp.maximum(jnp.maximum(y[0], y[1]), y[2])
        y = y[:30].reshape(10, 3, bt, 256)
        y = jnp.maximum(jnp.maximum(y[:, 0], y[:, 1]), y[:, 2])
        lo.append(y[..., :128])
        hi.append(y[..., 128:])
    o_ref[...] = jnp.concatenate(lo + hi, axis=0).astype(o_ref.dtype)


def _stage2_kernel(x_ref, w2_ref, b2_ref, w3_ref, b3_ref, wh_ref, bh_ref,
                   o_ref):
    bt = x_ref.shape[1]
    offs = [(t // 3) * 10 + t % 3 for t in range(9)]
    xs2 = jnp.concatenate(
        [x_ref[o:o + 78].reshape(78 * bt, 128) for o in offs], axis=1)
    acc2 = jnp.dot(xs2, w2_ref[...], preferred_element_type=jnp.float32)
    y2 = jnp.maximum(acc2 + b2_ref[...], 0.0).astype(x_ref.dtype)
    y2 = y2.reshape(78, bt, 256)

    xs3 = jnp.concatenate(
        [y2[o:o + 56].reshape(56 * bt, 256) for o in offs], axis=1)
    acc3 = jnp.dot(xs3, w3_ref[...], preferred_element_type=jnp.float32)
    y3 = jnp.maximum(acc3 + b3_ref[...], 0.0).astype(x_ref.dtype)
    y3 = y3.reshape(56, bt, 384)

    feats = jnp.concatenate(
        [y3[oh * 10 + ow] for oh in range(6) for ow in range(6)], axis=1)
    out = jnp.dot(feats, wh_ref[...], preferred_element_type=jnp.float32)
    o_ref[...] = out + bh_ref[...]


def _forward(xt, conv1_w, conv1_b, conv2_w, conv2_b, conv3_w, conv3_b,
             head_w, head_b):
    B = xt.shape[2]
    bt1 = 128 if B % 128 == 0 else B
    bt2 = 64 if B % 64 == 0 else B

    x1 = jnp.pad(xt, ((0, 0), (0, 2), (0, 0)))

    w1 = jnp.zeros((54, 256), jnp.float32)
    w1 = w1.at[:27, :128].set(conv1_w).at[27:, 128:].set(conv1_w).astype(_DT)
    b1 = jnp.concatenate([conv1_b, conv1_b], axis=1)

    pooled = pl.pallas_call(
        _conv1_pool_kernel,
        out_shape=jax.ShapeDtypeStruct((100, B, 128), _DT),
        grid=(B // bt1,),
        in_specs=[
            pl.BlockSpec((3, 1026, bt1), lambda i: (0, 0, i)),
            pl.BlockSpec((54, 256), lambda i: (0, 0)),
            pl.BlockSpec((1, 256), lambda i: (0, 0)),
        ],
        out_specs=pl.BlockSpec((100, bt1, 128), lambda i: (0, i, 0)),
        compiler_params=pltpu.CompilerParams(
            dimension_semantics=("parallel",), vmem_limit_bytes=_VMEM),
    )(x1, w1, b1)

    w2r = conv2_w.astype(_DT)
    w3r = jnp.pad(conv3_w, ((0, 0), (0, 84))).astype(_DT)
    b3p = jnp.pad(conv3_b, ((0, 0), (0, 84)))
    whr = jnp.pad(head_w.reshape(36, 300, 10),
                  ((0, 0), (0, 84), (0, 0))).reshape(36 * 384, 10).astype(_DT)

    out = pl.pallas_call(
        _stage2_kernel,
        out_shape=jax.ShapeDtypeStruct((B, 10), jnp.float32),
        grid=(B // bt2,),
        in_specs=[
            pl.BlockSpec((100, bt2, 128), lambda i: (0, i, 0)),
            pl.BlockSpec((1152, 256), lambda i: (0, 0)),
            pl.BlockSpec((1, 256), lambda i: (0, 0)),
            pl.BlockSpec((2304, 384), lambda i: (0, 0)),
            pl.BlockSpec((1, 384), lambda i: (0, 0)),
            pl.BlockSpec((36 * 384, 10), lambda i: (0, 0)),
            pl.BlockSpec((1, 10), lambda i: (0, 0)),
        ],
        out_specs=pl.BlockSpec((bt2, 10), lambda i: (i, 0)),
        compiler_params=pltpu.CompilerParams(
            dimension_semantics=("parallel",), vmem_limit_bytes=_VMEM),
    )(pooled, w2r, conv2_b, w3r, b3p, whr, head_b)
    return out


def kernel(x_nchw, conv1_w, conv1_b, conv2_w, conv2_b, conv3_w, conv3_b,
           head_w, head_b, sel1, sel2, sel3):
    B = x_nchw.shape[0]
    xt = jnp.transpose(x_nchw.reshape(B, 3 * 1024).astype(_DT))
    xt = xt.reshape(3, 1024, B)
    return _forward(xt, conv1_w, conv1_b, conv2_w, conv2_b, conv3_w,
                    conv3_b, head_w, head_b)
